# Initial kernel scaffold; baseline (speedup 1.0000x reference)
import jax
import jax.numpy as jnp
from jax.experimental import pallas as pl


def kernel(x, edge_index, W1, b1, W2, b2):
    raise NotImplementedError("write your pallas kernel here")


# speedup vs baseline: 5.6619x; 5.6619x over previous
import functools

import jax
import jax.numpy as jnp
from jax import lax
from jax.experimental import pallas as pl
from jax.experimental.pallas import tpu as pltpu
from jax.experimental.pallas import tpu_sc as plsc

_NC = 2
_NS = 16
_CHUNK = 128


def _sc_mesh():
    return plsc.VectorSubcoreMesh(core_axis_name="c", subcore_axis_name="s")


def _deg_partials(dst_pad, ones16, zer16, n_pad, e_pad):
    per_tile = e_pad // (_NC * _NS)
    n_chunks = per_tile // _CHUNK
    rows_pt = n_pad // _NS

    @functools.partial(
        pl.kernel,
        out_type=jax.ShapeDtypeStruct((2 * n_pad, 128), jnp.float32),
        mesh=_sc_mesh(),
        scratch_types=(pltpu.VMEM((_CHUNK, 128), jnp.float32),
                       pltpu.VMEM((_CHUNK,), jnp.int32),
                       pltpu.VMEM_SHARED((n_pad, 128), jnp.float32)),
    )
    def run(dst_hbm, ones_hbm, zer_hbm, degp_hbm, ones_v, idx_v, deg_sh):
        c = lax.axis_index("c")
        s = lax.axis_index("s")
        r0 = s * rows_pt
        pltpu.sync_copy(zer_hbm.at[pl.ds(r0, rows_pt)],
                        deg_sh.at[pl.ds(r0, rows_pt)])
        pltpu.sync_copy(ones_hbm, ones_v)
        plsc.subcore_barrier()

        base0 = (c * _NS + s) * per_tile

        @pl.loop(0, n_chunks)
        def _(i):
            base = base0 + i * _CHUNK
            pltpu.sync_copy(dst_hbm.at[pl.ds(base, _CHUNK)], idx_v)
            pltpu.sync_copy(ones_v, deg_sh.at[idx_v], add=True)

        plsc.subcore_barrier()
        pltpu.sync_copy(deg_sh.at[pl.ds(r0, rows_pt)],
                        degp_hbm.at[pl.ds(c * n_pad + r0, rows_pt)])

    degp = run(dst_pad, ones16, zer16)
    return degp[:n_pad], degp[n_pad:]


def _edge_scatter(src_pad, dst_pad, ya, yb, zer, n_pad, e_pad, half):
    per_tile = e_pad // _NS
    n_chunks = per_tile // _CHUNK
    rows_pt = n_pad // _NS

    @functools.partial(
        pl.kernel,
        out_type=(jax.ShapeDtypeStruct((n_pad, half), jnp.float32),
                  jax.ShapeDtypeStruct((n_pad, half), jnp.float32)),
        mesh=_sc_mesh(),
        scratch_types=(pltpu.VMEM((_CHUNK,), jnp.int32),
                       pltpu.VMEM((_CHUNK,), jnp.int32),
                       pltpu.VMEM((_CHUNK, half), jnp.float32),
                       pltpu.SemaphoreType.DMA,
                       pltpu.VMEM_SHARED((n_pad, half), jnp.float32)),
    )
    def run(src_hbm, dst_hbm, ya_hbm, yb_hbm, zer_hbm, acca_hbm, accb_hbm,
            idx_s, idx_d, rows_v, gsem, acc_sh):
        c = lax.axis_index("c")
        s = lax.axis_index("s")
        r0 = s * rows_pt
        pltpu.sync_copy(zer_hbm.at[pl.ds(r0, rows_pt)],
                        acc_sh.at[pl.ds(r0, rows_pt)])
        plsc.subcore_barrier()

        base0 = s * per_tile

        def edge_loop(ytab_hbm):
            @pl.loop(0, n_chunks)
            def _(i):
                base = base0 + i * _CHUNK
                pltpu.sync_copy(src_hbm.at[pl.ds(base, _CHUNK)], idx_s)
                pltpu.async_copy(ytab_hbm.at[idx_s], rows_v, gsem).wait()
                pltpu.sync_copy(dst_hbm.at[pl.ds(base, _CHUNK)], idx_d)
                pltpu.sync_copy(rows_v, acc_sh.at[idx_d], add=True)

        @pl.when(c == 0)
        def _():
            edge_loop(ya_hbm)

        @pl.when(c == 1)
        def _():
            edge_loop(yb_hbm)

        plsc.subcore_barrier()

        @pl.when(c == 0)
        def _():
            pltpu.sync_copy(acc_sh.at[pl.ds(r0, rows_pt)],
                            acca_hbm.at[pl.ds(r0, rows_pt)])

        @pl.when(c == 1)
        def _():
            pltpu.sync_copy(acc_sh.at[pl.ds(r0, rows_pt)],
                            accb_hbm.at[pl.ds(r0, rows_pt)])

    return run(src_pad, dst_pad, ya, yb, zer)


def _dis_block(dega_ref, degb_ref):
    deg = 1.0 + dega_ref[:, 0:1] + degb_ref[:, 0:1]
    return lax.rsqrt(deg)


def _mm_scale(x, w1, dega, degb, n, d, half, blk):
    grid = n // blk

    def body(x_ref, w_ref, dega_ref, degb_ref, ya_ref, yb_ref):
        dis = _dis_block(dega_ref, degb_ref)
        xw = jnp.dot(x_ref[...], w_ref[...],
                     preferred_element_type=jnp.float32)
        y = xw * dis
        ya_ref[...] = y[:, :half]
        yb_ref[...] = y[:, half:]

    return pl.pallas_call(
        body,
        grid=(grid,),
        in_specs=[
            pl.BlockSpec((blk, d), lambda i: (i, 0)),
            pl.BlockSpec((d, d), lambda i: (0, 0)),
            pl.BlockSpec((blk, 128), lambda i: (i, 0)),
            pl.BlockSpec((blk, 128), lambda i: (i, 0)),
        ],
        out_specs=[
            pl.BlockSpec((blk, half), lambda i: (i, 0)),
            pl.BlockSpec((blk, half), lambda i: (i, 0)),
        ],
        out_shape=[jax.ShapeDtypeStruct((n, half), jnp.float32),
                   jax.ShapeDtypeStruct((n, half), jnp.float32)],
    )(x, w1, dega, degb)


def _layer2_mm(acca, accb, ya, yb, dega, degb, b1r, w2, n, d, half, blk):
    grid = n // blk

    def body(acca_ref, accb_ref, ya_ref, yb_ref, dega_ref, degb_ref,
             b_ref, w_ref, y2a_ref, y2b_ref):
        dis = _dis_block(dega_ref, degb_ref)
        b = b_ref[...]
        ha = (acca_ref[...] + ya_ref[...]) * dis + b[:, :half]
        hb = (accb_ref[...] + yb_ref[...]) * dis + b[:, half:]
        h = jax.nn.relu(jnp.concatenate([ha, hb], axis=1))
        xw = jnp.dot(h, w_ref[...], preferred_element_type=jnp.float32)
        y2 = xw * dis
        y2a_ref[...] = y2[:, :half]
        y2b_ref[...] = y2[:, half:]

    return pl.pallas_call(
        body,
        grid=(grid,),
        in_specs=[
            pl.BlockSpec((blk, half), lambda i: (i, 0)),
            pl.BlockSpec((blk, half), lambda i: (i, 0)),
            pl.BlockSpec((blk, half), lambda i: (i, 0)),
            pl.BlockSpec((blk, half), lambda i: (i, 0)),
            pl.BlockSpec((blk, 128), lambda i: (i, 0)),
            pl.BlockSpec((blk, 128), lambda i: (i, 0)),
            pl.BlockSpec((1, d), lambda i: (0, 0)),
            pl.BlockSpec((d, d), lambda i: (0, 0)),
        ],
        out_specs=[
            pl.BlockSpec((blk, half), lambda i: (i, 0)),
            pl.BlockSpec((blk, half), lambda i: (i, 0)),
        ],
        out_shape=[jax.ShapeDtypeStruct((n, half), jnp.float32),
                   jax.ShapeDtypeStruct((n, half), jnp.float32)],
    )(acca, accb, ya, yb, dega, degb, b1r, w2)


def _final_combine(acca, accb, y2a, y2b, dega, degb, b2r, n, d, half, blk):
    grid = n // blk

    def body(acca_ref, accb_ref, ya_ref, yb_ref, dega_ref, degb_ref,
             b_ref, out_ref):
        dis = _dis_block(dega_ref, degb_ref)
        b = b_ref[...]
        oa = (acca_ref[...] + ya_ref[...]) * dis + b[:, :half]
        ob = (accb_ref[...] + yb_ref[...]) * dis + b[:, half:]
        out_ref[...] = jnp.concatenate([oa, ob], axis=1)

    return pl.pallas_call(
        body,
        grid=(grid,),
        in_specs=[
            pl.BlockSpec((blk, half), lambda i: (i, 0)),
            pl.BlockSpec((blk, half), lambda i: (i, 0)),
            pl.BlockSpec((blk, half), lambda i: (i, 0)),
            pl.BlockSpec((blk, half), lambda i: (i, 0)),
            pl.BlockSpec((blk, 128), lambda i: (i, 0)),
            pl.BlockSpec((blk, 128), lambda i: (i, 0)),
            pl.BlockSpec((1, d), lambda i: (0, 0)),
        ],
        out_specs=pl.BlockSpec((blk, d), lambda i: (i, 0)),
        out_shape=jax.ShapeDtypeStruct((n, d), jnp.float32),
    )(acca, accb, y2a, y2b, dega, degb, b2r)


def kernel(x, edge_index, W1, b1, W2, b2):
    n, d = x.shape
    e = edge_index.shape[1]
    half = d // 2

    align = _NC * _NS * _CHUNK
    e_pad = -(-e // align) * align
    pad = e_pad - e
    n_pad = -(-(n + 16) // (8 * _NS)) * (8 * _NS)
    pad_dst = n + (jnp.arange(pad, dtype=jnp.int32) % 16)
    src_pad = jnp.concatenate(
        [edge_index[0], jnp.zeros((pad,), jnp.int32)])
    dst_pad = jnp.concatenate([edge_index[1], pad_dst])

    zer = jnp.zeros((n_pad, half), jnp.float32)
    ones_rows = jnp.ones((_CHUNK, 128), jnp.float32)
    b1r = b1.reshape(1, d)
    b2r = b2.reshape(1, d)

    blk = 2000 if n % 2000 == 0 else 8 * (n // 8)
    while n % blk:
        blk -= 8

    dega, degb = _deg_partials(dst_pad, ones_rows, zer, n_pad, e_pad)
    ya, yb = _mm_scale(x, W1, dega, degb, n, d, half, blk)
    acc1a, acc1b = _edge_scatter(src_pad, dst_pad, ya, yb, zer,
                                 n_pad, e_pad, half)
    y2a, y2b = _layer2_mm(acc1a, acc1b, ya, yb, dega, degb, b1r, W2,
                          n, d, half, blk)
    acc2a, acc2b = _edge_scatter(src_pad, dst_pad, y2a, y2b, zer,
                                 n_pad, e_pad, half)
    return _final_combine(acc2a, acc2b, y2a, y2b, dega, degb, b2r,
                          n, d, half, blk)


# speedup vs baseline: 8.7247x; 1.5409x over previous
import functools

import jax
import jax.numpy as jnp
from jax import lax
from jax.experimental import pallas as pl
from jax.experimental.pallas import tpu as pltpu
from jax.experimental.pallas import tpu_sc as plsc

_NC = 2
_NS = 16
_CHUNK = 128


def _sc_mesh():
    return plsc.VectorSubcoreMesh(core_axis_name="c", subcore_axis_name="s")


def _deg_partials(dst_pad, ones16, zer16, n_pad, e_pad):
    per_tile = e_pad // (_NC * _NS)
    n_chunks = per_tile // _CHUNK
    rows_pt = n_pad // _NS

    @functools.partial(
        pl.kernel,
        out_type=jax.ShapeDtypeStruct((2 * n_pad, 128), jnp.float32),
        mesh=_sc_mesh(),
        scratch_types=(pltpu.VMEM((_CHUNK, 128), jnp.float32),
                       pltpu.VMEM((n_chunks, _CHUNK), jnp.int32),
                       pltpu.SemaphoreType.DMA,
                       pltpu.VMEM_SHARED((n_pad, 128), jnp.float32)),
    )
    def run(dst_hbm, ones_hbm, zer_hbm, degp_hbm, ones_v, didx, ssem, deg_sh):
        c = lax.axis_index("c")
        s = lax.axis_index("s")
        r0 = s * rows_pt
        crow0 = (c * _NS + s) * n_chunks
        pltpu.sync_copy(dst_hbm.at[pl.ds(crow0, n_chunks)], didx)
        pltpu.sync_copy(zer_hbm.at[pl.ds(r0, rows_pt)],
                        deg_sh.at[pl.ds(r0, rows_pt)])
        pltpu.sync_copy(ones_hbm, ones_v)
        plsc.subcore_barrier()

        @pl.loop(0, n_chunks)
        def _(k):
            pltpu.make_async_copy(ones_v, deg_sh.at[didx.at[k]],
                                  ssem).start(add=True)

        @pl.loop(0, n_chunks)
        def _(k):
            pltpu.make_async_copy(ones_v, deg_sh.at[didx.at[0]], ssem).wait()

        plsc.subcore_barrier()
        pltpu.sync_copy(deg_sh.at[pl.ds(r0, rows_pt)],
                        degp_hbm.at[pl.ds(c * n_pad + r0, rows_pt)])

    degp = run(dst_pad, ones16, zer16)
    return degp[:n_pad], degp[n_pad:]


def _edge_scatter(src_pad, dst_pad, ya, yb, zer, n_pad, e_pad, half):
    n_chunks = e_pad // (_NS * _CHUNK)
    rows_pt = n_pad // _NS
    nbuf = 2
    assert n_chunks % nbuf == 0

    @functools.partial(
        pl.kernel,
        out_type=(jax.ShapeDtypeStruct((n_pad, half), jnp.float32),
                  jax.ShapeDtypeStruct((n_pad, half), jnp.float32)),
        mesh=_sc_mesh(),
        scratch_types=(pltpu.VMEM((n_chunks, _CHUNK), jnp.int32),
                       pltpu.VMEM((_CHUNK, half), jnp.float32),
                       pltpu.VMEM((_CHUNK, half), jnp.float32),
                       pltpu.VMEM((_CHUNK,), jnp.int32),
                       pltpu.VMEM((_CHUNK,), jnp.int32),
                       pltpu.SemaphoreType.DMA,
                       pltpu.SemaphoreType.DMA,
                       pltpu.SemaphoreType.DMA,
                       pltpu.SemaphoreType.DMA,
                       pltpu.SemaphoreType.DMA,
                       pltpu.SemaphoreType.DMA,
                       pltpu.VMEM_SHARED((n_pad, half), jnp.float32)),
    )
    def run(src_hbm, dst_hbm, ya_hbm, yb_hbm, zer_hbm, acca_hbm, accb_hbm,
            sidx, b0, b1, di0, di1, g0, g1, s0, s1, e0, e1, acc_sh):
        c = lax.axis_index("c")
        s = lax.axis_index("s")
        r0 = s * rows_pt
        crow0 = s * n_chunks
        bufs = (b0, b1)
        didxs = (di0, di1)
        gsems = (g0, g1)
        ssems = (s0, s1)
        dsems = (e0, e1)
        pltpu.sync_copy(src_hbm.at[pl.ds(crow0, n_chunks)], sidx)
        pltpu.sync_copy(zer_hbm.at[pl.ds(r0, rows_pt)],
                        acc_sh.at[pl.ds(r0, rows_pt)])
        plsc.subcore_barrier()

        def pipeline(ytab_hbm):
            def gather_start(k, b):
                pltpu.make_async_copy(ytab_hbm.at[sidx.at[k]], bufs[b],
                                      gsems[b]).start()

            def gather_wait(b):
                pltpu.make_async_copy(ytab_hbm.at[sidx.at[0]], bufs[b],
                                      gsems[b]).wait()

            def didx_start(k, b):
                pltpu.make_async_copy(dst_hbm.at[crow0 + k], didxs[b],
                                      dsems[b]).start()

            def didx_wait(b):
                pltpu.make_async_copy(dst_hbm.at[crow0], didxs[b],
                                      dsems[b]).wait()

            def scat_start(b):
                pltpu.make_async_copy(bufs[b], acc_sh.at[didxs[b]],
                                      ssems[b]).start(add=True)

            def scat_wait(b):
                pltpu.make_async_copy(bufs[b], acc_sh.at[didxs[b]],
                                      ssems[b]).wait()

            for b in range(nbuf):
                gather_start(b, b)
                didx_start(b, b)

            @pl.loop(0, n_chunks, step=nbuf)
            def _(i):
                for b in range(nbuf):
                    k = i + b
                    gather_wait(b)
                    didx_wait(b)
                    scat_start(b)

                    @pl.when(k + nbuf < n_chunks)
                    def _():
                        scat_wait(b)
                        gather_start(k + nbuf, b)
                        didx_start(k + nbuf, b)

            for b in range(nbuf):
                scat_wait(b)

        @pl.when(c == 0)
        def _():
            pipeline(ya_hbm)

        @pl.when(c == 1)
        def _():
            pipeline(yb_hbm)

        plsc.subcore_barrier()

        @pl.when(c == 0)
        def _():
            pltpu.sync_copy(acc_sh.at[pl.ds(r0, rows_pt)],
                            acca_hbm.at[pl.ds(r0, rows_pt)])

        @pl.when(c == 1)
        def _():
            pltpu.sync_copy(acc_sh.at[pl.ds(r0, rows_pt)],
                            accb_hbm.at[pl.ds(r0, rows_pt)])

    return run(src_pad, dst_pad, ya, yb, zer)


def _dis_block(dega_ref, degb_ref):
    deg = 1.0 + dega_ref[:, 0:1] + degb_ref[:, 0:1]
    return lax.rsqrt(deg)


def _mm_scale(x, w1, dega, degb, n, d, half, blk):
    grid = n // blk

    def body(x_ref, w_ref, dega_ref, degb_ref, ya_ref, yb_ref):
        dis = _dis_block(dega_ref, degb_ref)
        xw = jnp.dot(x_ref[...], w_ref[...],
                     preferred_element_type=jnp.float32)
        y = xw * dis
        ya_ref[...] = y[:, :half]
        yb_ref[...] = y[:, half:]

    return pl.pallas_call(
        body,
        grid=(grid,),
        in_specs=[
            pl.BlockSpec((blk, d), lambda i: (i, 0)),
            pl.BlockSpec((d, d), lambda i: (0, 0)),
            pl.BlockSpec((blk, 128), lambda i: (i, 0)),
            pl.BlockSpec((blk, 128), lambda i: (i, 0)),
        ],
        out_specs=[
            pl.BlockSpec((blk, half), lambda i: (i, 0)),
            pl.BlockSpec((blk, half), lambda i: (i, 0)),
        ],
        out_shape=[jax.ShapeDtypeStruct((n, half), jnp.float32),
                   jax.ShapeDtypeStruct((n, half), jnp.float32)],
    )(x, w1, dega, degb)


def _layer2_mm(acca, accb, ya, yb, dega, degb, b1r, w2, n, d, half, blk):
    grid = n // blk

    def body(acca_ref, accb_ref, ya_ref, yb_ref, dega_ref, degb_ref,
             b_ref, w_ref, y2a_ref, y2b_ref):
        dis = _dis_block(dega_ref, degb_ref)
        b = b_ref[...]
        ha = (acca_ref[...] + ya_ref[...]) * dis + b[:, :half]
        hb = (accb_ref[...] + yb_ref[...]) * dis + b[:, half:]
        h = jax.nn.relu(jnp.concatenate([ha, hb], axis=1))
        xw = jnp.dot(h, w_ref[...], preferred_element_type=jnp.float32)
        y2 = xw * dis
        y2a_ref[...] = y2[:, :half]
        y2b_ref[...] = y2[:, half:]

    return pl.pallas_call(
        body,
        grid=(grid,),
        in_specs=[
            pl.BlockSpec((blk, half), lambda i: (i, 0)),
            pl.BlockSpec((blk, half), lambda i: (i, 0)),
            pl.BlockSpec((blk, half), lambda i: (i, 0)),
            pl.BlockSpec((blk, half), lambda i: (i, 0)),
            pl.BlockSpec((blk, 128), lambda i: (i, 0)),
            pl.BlockSpec((blk, 128), lambda i: (i, 0)),
            pl.BlockSpec((1, d), lambda i: (0, 0)),
            pl.BlockSpec((d, d), lambda i: (0, 0)),
        ],
        out_specs=[
            pl.BlockSpec((blk, half), lambda i: (i, 0)),
            pl.BlockSpec((blk, half), lambda i: (i, 0)),
        ],
        out_shape=[jax.ShapeDtypeStruct((n, half), jnp.float32),
                   jax.ShapeDtypeStruct((n, half), jnp.float32)],
    )(acca, accb, ya, yb, dega, degb, b1r, w2)


def _final_combine(acca, accb, y2a, y2b, dega, degb, b2r, n, d, half, blk):
    grid = n // blk

    def body(acca_ref, accb_ref, ya_ref, yb_ref, dega_ref, degb_ref,
             b_ref, out_ref):
        dis = _dis_block(dega_ref, degb_ref)
        b = b_ref[...]
        oa = (acca_ref[...] + ya_ref[...]) * dis + b[:, :half]
        ob = (accb_ref[...] + yb_ref[...]) * dis + b[:, half:]
        out_ref[...] = jnp.concatenate([oa, ob], axis=1)

    return pl.pallas_call(
        body,
        grid=(grid,),
        in_specs=[
            pl.BlockSpec((blk, half), lambda i: (i, 0)),
            pl.BlockSpec((blk, half), lambda i: (i, 0)),
            pl.BlockSpec((blk, half), lambda i: (i, 0)),
            pl.BlockSpec((blk, half), lambda i: (i, 0)),
            pl.BlockSpec((blk, 128), lambda i: (i, 0)),
            pl.BlockSpec((blk, 128), lambda i: (i, 0)),
            pl.BlockSpec((1, d), lambda i: (0, 0)),
        ],
        out_specs=pl.BlockSpec((blk, d), lambda i: (i, 0)),
        out_shape=jax.ShapeDtypeStruct((n, d), jnp.float32),
    )(acca, accb, y2a, y2b, dega, degb, b2r)


def kernel(x, edge_index, W1, b1, W2, b2):
    n, d = x.shape
    e = edge_index.shape[1]
    half = d // 2

    align = 4 * _NC * _NS * _CHUNK
    e_pad = -(-e // align) * align
    pad = e_pad - e
    n_pad = -(-(n + 16) // (8 * _NS)) * (8 * _NS)
    pad_dst = n + (jnp.arange(pad, dtype=jnp.int32) % 16)
    src_pad = jnp.concatenate(
        [edge_index[0], jnp.zeros((pad,), jnp.int32)])
    dst_pad = jnp.concatenate([edge_index[1], pad_dst])
    src2d = src_pad.reshape(e_pad // _CHUNK, _CHUNK)
    dst2d = dst_pad.reshape(e_pad // _CHUNK, _CHUNK)

    zer = jnp.zeros((n_pad, half), jnp.float32)
    ones_rows = jnp.ones((_CHUNK, 128), jnp.float32)
    b1r = b1.reshape(1, d)
    b2r = b2.reshape(1, d)

    blk = 2000 if n % 2000 == 0 else 8 * (n // 8)
    while n % blk:
        blk -= 8

    dega, degb = _deg_partials(dst2d, ones_rows, zer, n_pad, e_pad)
    ya, yb = _mm_scale(x, W1, dega, degb, n, d, half, blk)
    acc1a, acc1b = _edge_scatter(src2d, dst2d, ya, yb, zer,
                                 n_pad, e_pad, half)
    y2a, y2b = _layer2_mm(acc1a, acc1b, ya, yb, dega, degb, b1r, W2,
                          n, d, half, blk)
    acc2a, acc2b = _edge_scatter(src2d, dst2d, y2a, y2b, zer,
                                 n_pad, e_pad, half)
    return _final_combine(acc2a, acc2b, y2a, y2b, dega, degb, b2r,
                          n, d, half, blk)


# speedup vs baseline: 8.8690x; 1.0165x over previous
import functools

import jax
import jax.numpy as jnp
from jax import lax
from jax.experimental import pallas as pl
from jax.experimental.pallas import tpu as pltpu
from jax.experimental.pallas import tpu_sc as plsc

_NC = 2
_NS = 16
_CHUNK = 128


def _sc_mesh():
    return plsc.VectorSubcoreMesh(core_axis_name="c", subcore_axis_name="s")


def _deg_partials(dst_pad, ones16, zer16, n_pad, e_pad):
    per_tile = e_pad // (_NC * _NS)
    n_chunks = per_tile // _CHUNK
    rows_pt = n_pad // _NS

    @functools.partial(
        pl.kernel,
        out_type=jax.ShapeDtypeStruct((2 * n_pad, 128), jnp.float32),
        mesh=_sc_mesh(),
        scratch_types=(pltpu.VMEM((_CHUNK, 128), jnp.float32),
                       pltpu.VMEM((n_chunks, _CHUNK), jnp.int32),
                       pltpu.SemaphoreType.DMA,
                       pltpu.VMEM_SHARED((n_pad, 128), jnp.float32)),
    )
    def run(dst_hbm, ones_hbm, zer_hbm, degp_hbm, ones_v, didx, ssem, deg_sh):
        c = lax.axis_index("c")
        s = lax.axis_index("s")
        r0 = s * rows_pt
        crow0 = (c * _NS + s) * n_chunks
        pltpu.sync_copy(dst_hbm.at[pl.ds(crow0, n_chunks)], didx)
        pltpu.sync_copy(zer_hbm.at[pl.ds(r0, rows_pt)],
                        deg_sh.at[pl.ds(r0, rows_pt)])
        pltpu.sync_copy(ones_hbm, ones_v)
        plsc.subcore_barrier()

        @pl.loop(0, n_chunks)
        def _(k):
            pltpu.make_async_copy(ones_v, deg_sh.at[didx.at[k]],
                                  ssem).start(add=True)

        @pl.loop(0, n_chunks)
        def _(k):
            pltpu.make_async_copy(ones_v, deg_sh.at[didx.at[0]], ssem).wait()

        plsc.subcore_barrier()
        pltpu.sync_copy(deg_sh.at[pl.ds(r0, rows_pt)],
                        degp_hbm.at[pl.ds(c * n_pad + r0, rows_pt)])

    degp = run(dst_pad, ones16, zer16)
    return degp[:n_pad], degp[n_pad:]


def _edge_scatter(src_pad, dst_pad, ya, yb, zer, n_pad, e_pad, half):
    n_chunks = e_pad // (_NS * _CHUNK)
    rows_pt = n_pad // _NS
    nbuf = 2
    assert n_chunks % nbuf == 0

    @functools.partial(
        pl.kernel,
        out_type=(jax.ShapeDtypeStruct((n_pad, half), jnp.float32),
                  jax.ShapeDtypeStruct((n_pad, half), jnp.float32)),
        mesh=_sc_mesh(),
        scratch_types=(pltpu.VMEM((n_chunks, _CHUNK), jnp.int32),
                       pltpu.VMEM((_CHUNK, half), jnp.float32),
                       pltpu.VMEM((_CHUNK, half), jnp.float32),
                       pltpu.VMEM((_CHUNK,), jnp.int32),
                       pltpu.VMEM((_CHUNK,), jnp.int32),
                       pltpu.SemaphoreType.DMA,
                       pltpu.SemaphoreType.DMA,
                       pltpu.SemaphoreType.DMA,
                       pltpu.SemaphoreType.DMA,
                       pltpu.SemaphoreType.DMA,
                       pltpu.SemaphoreType.DMA,
                       pltpu.VMEM_SHARED((n_pad, half), jnp.float32)),
    )
    def run(src_hbm, dst_hbm, ya_hbm, yb_hbm, zer_hbm, acca_hbm, accb_hbm,
            sidx, b0, b1, di0, di1, g0, g1, s0, s1, e0, e1, acc_sh):
        c = lax.axis_index("c")
        s = lax.axis_index("s")
        r0 = s * rows_pt
        crow0 = s * n_chunks
        bufs = (b0, b1)
        didxs = (di0, di1)
        gsems = (g0, g1)
        ssems = (s0, s1)
        dsems = (e0, e1)
        pltpu.sync_copy(src_hbm.at[pl.ds(crow0, n_chunks)], sidx)
        pltpu.sync_copy(zer_hbm.at[pl.ds(r0, rows_pt)],
                        acc_sh.at[pl.ds(r0, rows_pt)])
        plsc.subcore_barrier()

        def pipeline(ytab_hbm):
            def gather_start(k, b):
                pltpu.make_async_copy(ytab_hbm.at[sidx.at[k]], bufs[b],
                                      gsems[b]).start()

            def gather_wait(b):
                pltpu.make_async_copy(ytab_hbm.at[sidx.at[0]], bufs[b],
                                      gsems[b]).wait()

            def didx_start(k, b):
                pltpu.make_async_copy(dst_hbm.at[crow0 + k], didxs[b],
                                      dsems[b]).start()

            def didx_wait(b):
                pltpu.make_async_copy(dst_hbm.at[crow0], didxs[b],
                                      dsems[b]).wait()

            def scat_start(b):
                pltpu.make_async_copy(bufs[b], acc_sh.at[didxs[b]],
                                      ssems[b]).start(add=True)

            def scat_wait(b):
                pltpu.make_async_copy(bufs[b], acc_sh.at[didxs[b]],
                                      ssems[b]).wait()

            for b in range(nbuf):
                gather_start(b, b)
                didx_start(b, b)

            @pl.loop(0, n_chunks, step=nbuf)
            def _(i):
                for b in range(nbuf):
                    k = i + b
                    gather_wait(b)
                    didx_wait(b)
                    scat_start(b)

                    @pl.when(k + nbuf < n_chunks)
                    def _():
                        scat_wait(b)
                        gather_start(k + nbuf, b)
                        didx_start(k + nbuf, b)

            for b in range(nbuf):
                scat_wait(b)

        @pl.when(c == 0)
        def _():
            pipeline(ya_hbm)

        @pl.when(c == 1)
        def _():
            pipeline(yb_hbm)

        plsc.subcore_barrier()

        @pl.when(c == 0)
        def _():
            pltpu.sync_copy(acc_sh.at[pl.ds(r0, rows_pt)],
                            acca_hbm.at[pl.ds(r0, rows_pt)])

        @pl.when(c == 1)
        def _():
            pltpu.sync_copy(acc_sh.at[pl.ds(r0, rows_pt)],
                            accb_hbm.at[pl.ds(r0, rows_pt)])

    return run(src_pad, dst_pad, ya, yb, zer)


def _dis_block(dega_ref, degb_ref):
    deg = 1.0 + dega_ref[:, 0:1] + degb_ref[:, 0:1]
    return lax.rsqrt(deg)


def _mm_only(x, w1, n, d, blk):
    grid = n // blk

    def body(x_ref, w_ref, xw_ref):
        xw_ref[...] = jnp.dot(x_ref[...], w_ref[...],
                              preferred_element_type=jnp.float32)

    return pl.pallas_call(
        body,
        grid=(grid,),
        in_specs=[
            pl.BlockSpec((blk, d), lambda i: (i, 0)),
            pl.BlockSpec((d, d), lambda i: (0, 0)),
        ],
        out_specs=pl.BlockSpec((blk, d), lambda i: (i, 0)),
        out_shape=jax.ShapeDtypeStruct((n, d), jnp.float32),
    )(x, w1)


def _scale_split(xw, dega, degb, n, d, half, blk):
    grid = n // blk

    def body(xw_ref, dega_ref, degb_ref, ya_ref, yb_ref):
        dis = _dis_block(dega_ref, degb_ref)
        y = xw_ref[...] * dis
        ya_ref[...] = y[:, :half]
        yb_ref[...] = y[:, half:]

    return pl.pallas_call(
        body,
        grid=(grid,),
        in_specs=[
            pl.BlockSpec((blk, d), lambda i: (i, 0)),
            pl.BlockSpec((blk, 128), lambda i: (i, 0)),
            pl.BlockSpec((blk, 128), lambda i: (i, 0)),
        ],
        out_specs=[
            pl.BlockSpec((blk, half), lambda i: (i, 0)),
            pl.BlockSpec((blk, half), lambda i: (i, 0)),
        ],
        out_shape=[jax.ShapeDtypeStruct((n, half), jnp.float32),
                   jax.ShapeDtypeStruct((n, half), jnp.float32)],
    )(xw, dega, degb)


def _layer2_mm(acca, accb, ya, yb, dega, degb, b1r, w2, n, d, half, blk):
    grid = n // blk

    def body(acca_ref, accb_ref, ya_ref, yb_ref, dega_ref, degb_ref,
             b_ref, w_ref, y2a_ref, y2b_ref):
        dis = _dis_block(dega_ref, degb_ref)
        b = b_ref[...]
        ha = (acca_ref[...] + ya_ref[...]) * dis + b[:, :half]
        hb = (accb_ref[...] + yb_ref[...]) * dis + b[:, half:]
        h = jax.nn.relu(jnp.concatenate([ha, hb], axis=1))
        xw = jnp.dot(h, w_ref[...], preferred_element_type=jnp.float32)
        y2 = xw * dis
        y2a_ref[...] = y2[:, :half]
        y2b_ref[...] = y2[:, half:]

    return pl.pallas_call(
        body,
        grid=(grid,),
        in_specs=[
            pl.BlockSpec((blk, half), lambda i: (i, 0)),
            pl.BlockSpec((blk, half), lambda i: (i, 0)),
            pl.BlockSpec((blk, half), lambda i: (i, 0)),
            pl.BlockSpec((blk, half), lambda i: (i, 0)),
            pl.BlockSpec((blk, 128), lambda i: (i, 0)),
            pl.BlockSpec((blk, 128), lambda i: (i, 0)),
            pl.BlockSpec((1, d), lambda i: (0, 0)),
            pl.BlockSpec((d, d), lambda i: (0, 0)),
        ],
        out_specs=[
            pl.BlockSpec((blk, half), lambda i: (i, 0)),
            pl.BlockSpec((blk, half), lambda i: (i, 0)),
        ],
        out_shape=[jax.ShapeDtypeStruct((n, half), jnp.float32),
                   jax.ShapeDtypeStruct((n, half), jnp.float32)],
    )(acca, accb, ya, yb, dega, degb, b1r, w2)


def _final_combine(acca, accb, y2a, y2b, dega, degb, b2r, n, d, half, blk):
    grid = n // blk

    def body(acca_ref, accb_ref, ya_ref, yb_ref, dega_ref, degb_ref,
             b_ref, out_ref):
        dis = _dis_block(dega_ref, degb_ref)
        b = b_ref[...]
        oa = (acca_ref[...] + ya_ref[...]) * dis + b[:, :half]
        ob = (accb_ref[...] + yb_ref[...]) * dis + b[:, half:]
        out_ref[...] = jnp.concatenate([oa, ob], axis=1)

    return pl.pallas_call(
        body,
        grid=(grid,),
        in_specs=[
            pl.BlockSpec((blk, half), lambda i: (i, 0)),
            pl.BlockSpec((blk, half), lambda i: (i, 0)),
            pl.BlockSpec((blk, half), lambda i: (i, 0)),
            pl.BlockSpec((blk, half), lambda i: (i, 0)),
            pl.BlockSpec((blk, 128), lambda i: (i, 0)),
            pl.BlockSpec((blk, 128), lambda i: (i, 0)),
            pl.BlockSpec((1, d), lambda i: (0, 0)),
        ],
        out_specs=pl.BlockSpec((blk, d), lambda i: (i, 0)),
        out_shape=jax.ShapeDtypeStruct((n, d), jnp.float32),
    )(acca, accb, y2a, y2b, dega, degb, b2r)


def kernel(x, edge_index, W1, b1, W2, b2):
    n, d = x.shape
    e = edge_index.shape[1]
    half = d // 2

    align = 4 * _NC * _NS * _CHUNK
    e_pad = -(-e // align) * align
    pad = e_pad - e
    n_pad = -(-(n + 16) // (8 * _NS)) * (8 * _NS)
    pad_dst = n + (jnp.arange(pad, dtype=jnp.int32) % 16)
    src_pad = jnp.concatenate(
        [edge_index[0], jnp.zeros((pad,), jnp.int32)])
    dst_pad = jnp.concatenate([edge_index[1], pad_dst])
    src2d = src_pad.reshape(e_pad // _CHUNK, _CHUNK)
    dst2d = dst_pad.reshape(e_pad // _CHUNK, _CHUNK)

    zer = jnp.zeros((n_pad, half), jnp.float32)
    ones_rows = jnp.ones((_CHUNK, 128), jnp.float32)
    b1r = b1.reshape(1, d)
    b2r = b2.reshape(1, d)

    blk = 2000 if n % 2000 == 0 else 8 * (n // 8)
    while n % blk:
        blk -= 8

    xw1 = _mm_only(x, W1, n, d, blk)
    dega, degb = _deg_partials(dst2d, ones_rows, zer, n_pad, e_pad)
    ya, yb = _scale_split(xw1, dega, degb, n, d, half, blk)
    acc1a, acc1b = _edge_scatter(src2d, dst2d, ya, yb, zer,
                                 n_pad, e_pad, half)
    y2a, y2b = _layer2_mm(acc1a, acc1b, ya, yb, dega, degb, b1r, W2,
                          n, d, half, blk)
    acc2a, acc2b = _edge_scatter(src2d, dst2d, y2a, y2b, zer,
                                 n_pad, e_pad, half)
    return _final_combine(acc2a, acc2b, y2a, y2b, dega, degb, b2r,
                          n, d, half, blk)
